# baseline (speedup 1.0000x reference)
import jax
import jax.numpy as jnp
from jax.experimental import pallas as pl
from jax.experimental.pallas import tpu as pltpu

_B, _N, _D = 8, 1024, 64
_K = 8192
_HEADS = 4
_QK_DIM = 64

_TT = 1024
_KT = 1024
_NT = (_B * _N) // _TT
_NK = _K // _KT


def _l2norm_rows(x):
    n = jnp.sqrt(jnp.sum(x * x, axis=-1, keepdims=True))
    return x / jnp.maximum(n, 1e-12)


def _assign_body(x_ref, m_ref, xn_ref, bkt_ref, best_ref, bidx_ref):
    j = pl.program_id(1)
    xn = _l2norm_rows(x_ref[...])
    mn = _l2norm_rows(m_ref[...])
    dist = jax.lax.dot_general(
        mn, xn, (((1,), (1,)), ((), ())),
        preferred_element_type=jnp.float32,
        precision=jax.lax.Precision.DEFAULT)
    maxv = jnp.max(dist, axis=0)
    gidx = jax.lax.broadcasted_iota(jnp.int32, dist.shape, 0) + j * _KT
    cand = jnp.min(jnp.where(dist == maxv[None, :], gidx, jnp.int32(_K)),
                   axis=0)

    @pl.when(j == 0)
    def _():
        best_ref[0, :] = maxv
        bidx_ref[0, :] = cand

    @pl.when(j > 0)
    def _():
        bv = best_ref[0, :]
        upd = maxv > bv
        best_ref[0, :] = jnp.where(upd, maxv, bv)
        bidx_ref[0, :] = jnp.where(upd, cand, bidx_ref[0, :])

    xn_ref[...] = xn

    @pl.when(j == _NK - 1)
    def _():
        bkt_ref[0, 0, :] = bidx_ref[0, :]


def _reduce_body(xn_ref, bkt_ref, m_ref, wk_ref, wv_ref,
                 xg_ref, k_ref, v_ref, acc_ref):
    j = pl.program_id(0)
    t = pl.program_id(1)

    @pl.when(t == 0)
    def _():
        acc_ref[...] = jnp.zeros_like(acc_ref)

    b = bkt_ref[0, 0, :]
    iota_c = jax.lax.broadcasted_iota(jnp.int32, (_KT, _TT), 0) + j * _KT
    oh = (iota_c == b[None, :]).astype(jnp.float32)
    xn = xn_ref[...]
    xa = jnp.concatenate([xn, jnp.ones((_TT, 1), jnp.float32)], axis=1)
    acc_ref[...] += jax.lax.dot_general(
        oh, xa, (((1,), (0,)), ((), ())),
        preferred_element_type=jnp.float32,
        precision=jax.lax.Precision.HIGHEST)

    @pl.when(t == _NT - 1)
    def _():
        acc = acc_ref[...]
        sums = acc[:, :_D]
        cnt = acc[:, _D:_D + 1]
        mn = _l2norm_rows(m_ref[...])
        xg = jnp.where(cnt == 0.0, mn, _l2norm_rows(sums))
        xg_ref[...] = xg
        k_ref[...] = jax.lax.dot_general(
            xg, wk_ref[...], (((1,), (1,)), ((), ())),
            preferred_element_type=jnp.float32,
            precision=jax.lax.Precision.HIGHEST)
        v_ref[...] = jax.lax.dot_general(
            xg, wv_ref[...], (((1,), (1,)), ((), ())),
            preferred_element_type=jnp.float32,
            precision=jax.lax.Precision.HIGHEST)


def kernel(normed_x, x_means, W_k, W_v):
    x = normed_x.reshape(_B * _N, _D)

    xn, buckets = pl.pallas_call(
        _assign_body,
        grid=(_NT, _NK),
        in_specs=[
            pl.BlockSpec((_TT, _D), lambda i, j: (i, 0)),
            pl.BlockSpec((_KT, _D), lambda i, j: (j, 0)),
        ],
        out_specs=[
            pl.BlockSpec((_TT, _D), lambda i, j: (i, 0)),
            pl.BlockSpec((1, 1, _TT), lambda i, j: (i, 0, 0)),
        ],
        out_shape=[
            jax.ShapeDtypeStruct((_B * _N, _D), jnp.float32),
            jax.ShapeDtypeStruct((_NT, 1, _TT), jnp.int32),
        ],
        scratch_shapes=[
            pltpu.VMEM((1, _TT), jnp.float32),
            pltpu.VMEM((1, _TT), jnp.int32),
        ],
        compiler_params=pltpu.CompilerParams(
            dimension_semantics=("arbitrary", "arbitrary")),
    )(x, x_means)

    xg, k, v = pl.pallas_call(
        _reduce_body,
        grid=(_NK, _NT),
        in_specs=[
            pl.BlockSpec((_TT, _D), lambda j, t: (t, 0)),
            pl.BlockSpec((1, 1, _TT), lambda j, t: (t, 0, 0)),
            pl.BlockSpec((_KT, _D), lambda j, t: (j, 0)),
            pl.BlockSpec((_QK_DIM, _D), lambda j, t: (0, 0)),
            pl.BlockSpec((_D, _D), lambda j, t: (0, 0)),
        ],
        out_specs=[
            pl.BlockSpec((_KT, _D), lambda j, t: (j, 0)),
            pl.BlockSpec((_KT, _QK_DIM), lambda j, t: (j, 0)),
            pl.BlockSpec((_KT, _D), lambda j, t: (j, 0)),
        ],
        out_shape=[
            jax.ShapeDtypeStruct((_K, _D), jnp.float32),
            jax.ShapeDtypeStruct((_K, _QK_DIM), jnp.float32),
            jax.ShapeDtypeStruct((_K, _D), jnp.float32),
        ],
        scratch_shapes=[
            pltpu.VMEM((_KT, _D + 1), jnp.float32),
        ],
        compiler_params=pltpu.CompilerParams(
            dimension_semantics=("arbitrary", "arbitrary")),
    )(xn, buckets, x_means, W_k, W_v)

    k = k.reshape(_K, _HEADS, _QK_DIM // _HEADS).transpose(1, 0, 2)
    v = v.reshape(_K, _HEADS, _D // _HEADS).transpose(1, 0, 2)
    return (k, v, jax.lax.stop_gradient(xg))
